# baseline (speedup 1.0000x reference)
import functools

import jax
import jax.numpy as jnp
from jax import lax
from jax.experimental import pallas as pl
from jax.experimental.pallas import tpu as pltpu

NN = 10000
EE = 160000
HH = 32
NODE_IN = 128
EDGE_IN = 16
NODE_OUT = 128

NB = 1000
TE = 1024
E_PAD = 163840
MSG_W = 40


def _proj1_body(x_ref, w_ref, b_ref, o_ref):
    o_ref[...] = (
        jnp.dot(x_ref[...], w_ref[...], preferred_element_type=jnp.float32)
        + b_ref[...]
    )


def _node_proj1(x, Wp1, bp1):
    return pl.pallas_call(
        _proj1_body,
        grid=(NN // NB,),
        in_specs=[
            pl.BlockSpec((NB, NODE_IN), lambda i: (i, 0)),
            pl.BlockSpec((NODE_IN, HH), lambda i: (0, 0)),
            pl.BlockSpec((1, HH), lambda i: (0, 0)),
        ],
        out_specs=pl.BlockSpec((NB, HH), lambda i: (i, 0)),
        out_shape=jax.ShapeDtypeStruct((NN, HH), jnp.float32),
    )(x, Wp1, bp1.reshape(1, HH))


def _edge_body(ea_ref, hs_ref, We1_ref, be1_ref, We2_ref, be2_ref, o_ref):
    eact = jnp.maximum(
        jnp.dot(ea_ref[...], We1_ref[...], preferred_element_type=jnp.float32)
        + be1_ref[...],
        0.0,
    )
    e_row = (
        jnp.dot(eact, We2_ref[...], preferred_element_type=jnp.float32)
        + be2_ref[...]
    )
    hs = hs_ref[...]
    acc = hs[:, 0:1] * e_row[:, 0:HH]
    for i in range(1, HH):
        acc = acc + hs[:, i : i + 1] * e_row[:, i * HH : (i + 1) * HH]
    ones = jnp.ones((TE, 1), jnp.float32)
    zeros = jnp.zeros((TE, MSG_W - HH - 1), jnp.float32)
    o_ref[...] = jnp.concatenate([acc, ones, zeros], axis=1)


def _edge_messages(ea_pad, h_src, We1, be1, We2, be2):
    return pl.pallas_call(
        _edge_body,
        grid=(E_PAD // TE,),
        in_specs=[
            pl.BlockSpec((TE, EDGE_IN), lambda i: (i, 0)),
            pl.BlockSpec((TE, HH), lambda i: (i, 0)),
            pl.BlockSpec((EDGE_IN, HH), lambda i: (0, 0)),
            pl.BlockSpec((1, HH), lambda i: (0, 0)),
            pl.BlockSpec((HH, HH * HH), lambda i: (0, 0)),
            pl.BlockSpec((1, HH * HH), lambda i: (0, 0)),
        ],
        out_specs=pl.BlockSpec((TE, MSG_W), lambda i: (i, 0)),
        out_shape=jax.ShapeDtypeStruct((E_PAD, MSG_W), jnp.float32),
    )(ea_pad, h_src, We1, be1.reshape(1, HH), We2, be2.reshape(1, HH * HH))


def _final_body(a_ref, h_ref, Wroot_ref, bconv_ref, Wp2_ref, bp2_ref, o_ref):
    a = a_ref[...]
    cnt = jnp.maximum(a[:, HH : HH + 1], 1.0)
    mean = a[:, :HH] / cnt
    hroot = jnp.dot(h_ref[...], Wroot_ref[...], preferred_element_type=jnp.float32)
    hh = jnp.maximum(mean + hroot + bconv_ref[...], 0.0)
    o_ref[...] = (
        jnp.dot(hh, Wp2_ref[...], preferred_element_type=jnp.float32) + bp2_ref[...]
    )


def _finalize(agg, h, Wroot, bconv, Wp2, bp2):
    return pl.pallas_call(
        _final_body,
        grid=(NN // NB,),
        in_specs=[
            pl.BlockSpec((NB, MSG_W), lambda i: (i, 0)),
            pl.BlockSpec((NB, HH), lambda i: (i, 0)),
            pl.BlockSpec((HH, HH), lambda i: (0, 0)),
            pl.BlockSpec((1, HH), lambda i: (0, 0)),
            pl.BlockSpec((HH, NODE_OUT), lambda i: (0, 0)),
            pl.BlockSpec((1, NODE_OUT), lambda i: (0, 0)),
        ],
        out_specs=pl.BlockSpec((NB, NODE_OUT), lambda i: (i, 0)),
        out_shape=jax.ShapeDtypeStruct((NN, NODE_OUT), jnp.float32),
    )(agg, h, Wroot, bconv.reshape(1, HH), Wp2, bp2.reshape(1, NODE_OUT))


def kernel(x, edge_index, edge_attr, Wp1, bp1, We1, be1, We2, be2, Wroot, bconv, Wp2, bp2):
    src = edge_index[0]
    dst = edge_index[1]
    npad = E_PAD - EE
    src_p = jnp.concatenate([src, jnp.zeros((npad,), jnp.int32)])
    dst_p = jnp.concatenate([dst, jnp.full((npad,), NN, jnp.int32)])
    ea_p = jnp.concatenate([edge_attr, jnp.zeros((npad, EDGE_IN), jnp.float32)])

    h = _node_proj1(x, Wp1, bp1)
    h_src = jnp.take(h, src_p, axis=0)
    msg = _edge_messages(ea_p, h_src, We1, be1, We2, be2)
    agg = jax.ops.segment_sum(msg, dst_p, num_segments=NN + 8)[:NN]
    return _finalize(agg, h, Wroot, bconv, Wp2, bp2)


# speedup vs baseline: 1.7428x; 1.7428x over previous
import functools

import jax
import jax.numpy as jnp
from jax import lax
from jax.experimental import pallas as pl
from jax.experimental.pallas import tpu as pltpu

NN = 10000
EE = 160000
HH = 32
NODE_IN = 128
EDGE_IN = 16
NODE_OUT = 128

NB = 1000
TE = 1024
E_PAD = 163840
MSG_W = 40


def _proj1_body(x_ref, w_ref, b_ref, o_ref):
    o_ref[...] = (
        jnp.dot(x_ref[...], w_ref[...], preferred_element_type=jnp.float32)
        + b_ref[...]
    )


def _node_proj1(x, Wp1, bp1):
    return pl.pallas_call(
        _proj1_body,
        grid=(NN // NB,),
        in_specs=[
            pl.BlockSpec((NB, NODE_IN), lambda i: (i, 0)),
            pl.BlockSpec((NODE_IN, HH), lambda i: (0, 0)),
            pl.BlockSpec((1, HH), lambda i: (0, 0)),
        ],
        out_specs=pl.BlockSpec((NB, HH), lambda i: (i, 0)),
        out_shape=jax.ShapeDtypeStruct((NN, HH), jnp.float32),
    )(x, Wp1, bp1.reshape(1, HH))


def _edge_body(ea_ref, hs_ref, We1_ref, be1_ref, We2_ref, be2_ref, rsel_ref, o_ref):
    eact = jnp.maximum(
        jnp.dot(ea_ref[...], We1_ref[...], preferred_element_type=jnp.float32)
        + be1_ref[...],
        0.0,
    )
    e_row = (
        jnp.dot(eact, We2_ref[...], preferred_element_type=jnp.float32)
        + be2_ref[...]
    )
    w = jnp.dot(hs_ref[...], rsel_ref[...], preferred_element_type=jnp.float32)
    acc = e_row[:, 0:128] * w[:, 0:128]
    for g in range(1, (HH * HH) // 128):
        acc = acc + e_row[:, g * 128 : (g + 1) * 128] * w[:, g * 128 : (g + 1) * 128]
    msg = acc[:, 0:HH] + acc[:, HH : 2 * HH] + acc[:, 2 * HH : 3 * HH] + acc[:, 3 * HH : 4 * HH]
    ones = jnp.ones((TE, 1), jnp.float32)
    zeros = jnp.zeros((TE, MSG_W - HH - 1), jnp.float32)
    o_ref[...] = jnp.concatenate([msg, ones, zeros], axis=1)


def _edge_messages(ea_pad, h_src, We1, be1, We2, be2):
    rsel = jnp.kron(jnp.eye(HH, dtype=jnp.float32), jnp.ones((1, HH), jnp.float32))
    return pl.pallas_call(
        _edge_body,
        grid=(E_PAD // TE,),
        in_specs=[
            pl.BlockSpec((TE, EDGE_IN), lambda i: (i, 0)),
            pl.BlockSpec((TE, HH), lambda i: (i, 0)),
            pl.BlockSpec((EDGE_IN, HH), lambda i: (0, 0)),
            pl.BlockSpec((1, HH), lambda i: (0, 0)),
            pl.BlockSpec((HH, HH * HH), lambda i: (0, 0)),
            pl.BlockSpec((1, HH * HH), lambda i: (0, 0)),
            pl.BlockSpec((HH, HH * HH), lambda i: (0, 0)),
        ],
        out_specs=pl.BlockSpec((TE, MSG_W), lambda i: (i, 0)),
        out_shape=jax.ShapeDtypeStruct((E_PAD, MSG_W), jnp.float32),
    )(ea_pad, h_src, We1, be1.reshape(1, HH), We2, be2.reshape(1, HH * HH), rsel)


def _final_body(a_ref, h_ref, Wroot_ref, bconv_ref, Wp2_ref, bp2_ref, o_ref):
    a = a_ref[...]
    cnt = jnp.maximum(a[:, HH : HH + 1], 1.0)
    mean = a[:, :HH] / cnt
    hroot = jnp.dot(h_ref[...], Wroot_ref[...], preferred_element_type=jnp.float32)
    hh = jnp.maximum(mean + hroot + bconv_ref[...], 0.0)
    o_ref[...] = (
        jnp.dot(hh, Wp2_ref[...], preferred_element_type=jnp.float32) + bp2_ref[...]
    )


def _finalize(agg, h, Wroot, bconv, Wp2, bp2):
    return pl.pallas_call(
        _final_body,
        grid=(NN // NB,),
        in_specs=[
            pl.BlockSpec((NB, MSG_W), lambda i: (i, 0)),
            pl.BlockSpec((NB, HH), lambda i: (i, 0)),
            pl.BlockSpec((HH, HH), lambda i: (0, 0)),
            pl.BlockSpec((1, HH), lambda i: (0, 0)),
            pl.BlockSpec((HH, NODE_OUT), lambda i: (0, 0)),
            pl.BlockSpec((1, NODE_OUT), lambda i: (0, 0)),
        ],
        out_specs=pl.BlockSpec((NB, NODE_OUT), lambda i: (i, 0)),
        out_shape=jax.ShapeDtypeStruct((NN, NODE_OUT), jnp.float32),
    )(agg, h, Wroot, bconv.reshape(1, HH), Wp2, bp2.reshape(1, NODE_OUT))


def kernel(x, edge_index, edge_attr, Wp1, bp1, We1, be1, We2, be2, Wroot, bconv, Wp2, bp2):
    src = edge_index[0]
    dst = edge_index[1]
    npad = E_PAD - EE
    src_p = jnp.concatenate([src, jnp.zeros((npad,), jnp.int32)])
    dst_p = jnp.concatenate([dst, jnp.full((npad,), NN, jnp.int32)])
    ea_p = jnp.concatenate([edge_attr, jnp.zeros((npad, EDGE_IN), jnp.float32)])

    h = _node_proj1(x, Wp1, bp1)
    h_src = jnp.take(h, src_p, axis=0)
    msg = _edge_messages(ea_p, h_src, We1, be1, We2, be2)
    agg = jax.ops.segment_sum(msg, dst_p, num_segments=NN + 8)[:NN]
    return _finalize(agg, h, Wroot, bconv, Wp2, bp2)


# speedup vs baseline: 4.7651x; 2.7342x over previous
import functools

import jax
import jax.numpy as jnp
from jax import lax
from jax.experimental import pallas as pl
from jax.experimental.pallas import tpu as pltpu
from jax.experimental.pallas import tpu_sc as plsc

NN = 10000
EE = 160000
HH = 32
NODE_IN = 128
EDGE_IN = 16
NODE_OUT = 128

NB = 1000
TE = 1024
E_PAD = 163840
MSG_W = 40
N_PAD = 10240

NC = 2
NS = 16
NW = NC * NS
IDXW = 128
ROWS_W = E_PAD // NW // IDXW
CHB = 8
CH = CHB * IDXW
NCHUNK = ROWS_W // CHB

_SC_MESH = plsc.VectorSubcoreMesh(core_axis_name="c", subcore_axis_name="s")


def _gather_body(h_hbm, src_hbm, out_hbm, h_sh, idx_v, rows_v, sem):
    core = lax.axis_index("c")
    sub = lax.axis_index("s")
    wid = sub * NC + core
    @pl.when(sub < 10)
    def _():
        pltpu.sync_copy(h_hbm.at[pl.ds(sub * 1000, 1000)],
                        h_sh.at[pl.ds(sub * 1000, 1000)])

    plsc.subcore_barrier()
    for c in range(NCHUNK):
        row0 = wid * ROWS_W + c * CHB
        pltpu.sync_copy(src_hbm.at[pl.ds(row0, CHB)], idx_v)
        cps = [
            pltpu.async_copy(
                h_sh.at[idx_v.at[j]], rows_v.at[pl.ds(j * IDXW, IDXW)], sem
            )
            for j in range(CHB)
        ]
        for cp in cps:
            cp.wait()
        pltpu.sync_copy(rows_v, out_hbm.at[pl.ds(row0 * IDXW, CH)])


def _sc_gather(h, src2d):
    return pl.kernel(
        _gather_body,
        out_type=jax.ShapeDtypeStruct((E_PAD, HH), jnp.float32),
        mesh=_SC_MESH,
        compiler_params=pltpu.CompilerParams(use_tc_tiling_on_sc=False),
        scratch_types=[
            pltpu.VMEM_SHARED((NN, HH), jnp.float32),
            pltpu.VMEM((CHB, IDXW), jnp.int32),
            pltpu.VMEM((CH, HH), jnp.float32),
            pltpu.SemaphoreType.DMA,
        ],
    )(h, src2d)


def _scatter_body(msg_hbm, dst_hbm, zero_hbm, out0_hbm, out1_hbm,
                  idx_v, rows_v, agg_sh, sem):
    core = lax.axis_index("c")
    sub = lax.axis_index("s")
    wid = sub * NC + core
    zrows = N_PAD // NS
    pltpu.sync_copy(zero_hbm.at[pl.ds(sub * zrows, zrows)],
                    agg_sh.at[pl.ds(sub * zrows, zrows)])
    plsc.subcore_barrier()
    for c in range(NCHUNK):
        row0 = wid * ROWS_W + c * CHB
        pltpu.sync_copy(dst_hbm.at[pl.ds(row0, CHB)], idx_v)
        pltpu.sync_copy(msg_hbm.at[pl.ds(row0 * IDXW, CH)], rows_v)
        cps = [
            pltpu.async_copy(
                rows_v.at[pl.ds(j * IDXW, IDXW)], agg_sh.at[idx_v.at[j]], sem,
                add=True,
            )
            for j in range(CHB)
        ]
        for cp in cps:
            cp.wait()
    plsc.subcore_barrier()

    @pl.when(core == 0)
    def _():
        pltpu.sync_copy(agg_sh.at[pl.ds(sub * zrows, zrows)],
                        out0_hbm.at[pl.ds(sub * zrows, zrows)])

    @pl.when(core == 1)
    def _():
        pltpu.sync_copy(agg_sh.at[pl.ds(sub * zrows, zrows)],
                        out1_hbm.at[pl.ds(sub * zrows, zrows)])


def _sc_scatter(msg, dst2d):
    zero = jnp.zeros((N_PAD, MSG_W), jnp.float32)
    return pl.kernel(
        _scatter_body,
        out_type=(
            jax.ShapeDtypeStruct((N_PAD, MSG_W), jnp.float32),
            jax.ShapeDtypeStruct((N_PAD, MSG_W), jnp.float32),
        ),
        mesh=_SC_MESH,
        compiler_params=pltpu.CompilerParams(use_tc_tiling_on_sc=False),
        scratch_types=[
            pltpu.VMEM((CHB, IDXW), jnp.int32),
            pltpu.VMEM((CH, MSG_W), jnp.float32),
            pltpu.VMEM_SHARED((N_PAD, MSG_W), jnp.float32),
            pltpu.SemaphoreType.DMA,
        ],
    )(msg, dst2d, zero)


def _proj1_body(x_ref, w_ref, b_ref, o_ref):
    o_ref[...] = (
        jnp.dot(x_ref[...], w_ref[...], preferred_element_type=jnp.float32)
        + b_ref[...]
    )


def _node_proj1(x, Wp1, bp1):
    return pl.pallas_call(
        _proj1_body,
        grid=(NN // NB,),
        in_specs=[
            pl.BlockSpec((NB, NODE_IN), lambda i: (i, 0)),
            pl.BlockSpec((NODE_IN, HH), lambda i: (0, 0)),
            pl.BlockSpec((1, HH), lambda i: (0, 0)),
        ],
        out_specs=pl.BlockSpec((NB, HH), lambda i: (i, 0)),
        out_shape=jax.ShapeDtypeStruct((NN, HH), jnp.float32),
    )(x, Wp1, bp1.reshape(1, HH))


def _edge_body(ea_ref, hs_ref, We1_ref, be1_ref, We2_ref, be2_ref, rsel_ref, o_ref):
    eact = jnp.maximum(
        jnp.dot(ea_ref[...], We1_ref[...], preferred_element_type=jnp.float32)
        + be1_ref[...],
        0.0,
    )
    e_row = (
        jnp.dot(eact, We2_ref[...], preferred_element_type=jnp.float32)
        + be2_ref[...]
    )
    w = jnp.dot(hs_ref[...], rsel_ref[...], preferred_element_type=jnp.float32)
    acc = e_row[:, 0:128] * w[:, 0:128]
    for g in range(1, (HH * HH) // 128):
        acc = acc + e_row[:, g * 128 : (g + 1) * 128] * w[:, g * 128 : (g + 1) * 128]
    msg = acc[:, 0:HH] + acc[:, HH : 2 * HH] + acc[:, 2 * HH : 3 * HH] + acc[:, 3 * HH : 4 * HH]
    ones = jnp.ones((TE, 1), jnp.float32)
    zeros = jnp.zeros((TE, MSG_W - HH - 1), jnp.float32)
    o_ref[...] = jnp.concatenate([msg, ones, zeros], axis=1)


def _edge_messages(ea_pad, h_src, We1, be1, We2, be2):
    rsel = jnp.kron(jnp.eye(HH, dtype=jnp.float32), jnp.ones((1, HH), jnp.float32))
    return pl.pallas_call(
        _edge_body,
        grid=(E_PAD // TE,),
        in_specs=[
            pl.BlockSpec((TE, EDGE_IN), lambda i: (i, 0)),
            pl.BlockSpec((TE, HH), lambda i: (i, 0)),
            pl.BlockSpec((EDGE_IN, HH), lambda i: (0, 0)),
            pl.BlockSpec((1, HH), lambda i: (0, 0)),
            pl.BlockSpec((HH, HH * HH), lambda i: (0, 0)),
            pl.BlockSpec((1, HH * HH), lambda i: (0, 0)),
            pl.BlockSpec((HH, HH * HH), lambda i: (0, 0)),
        ],
        out_specs=pl.BlockSpec((TE, MSG_W), lambda i: (i, 0)),
        out_shape=jax.ShapeDtypeStruct((E_PAD, MSG_W), jnp.float32),
    )(ea_pad, h_src, We1, be1.reshape(1, HH), We2, be2.reshape(1, HH * HH), rsel)


def _final_body(a_ref, b_ref, h_ref, Wroot_ref, bconv_ref, Wp2_ref, bp2_ref, o_ref):
    a = a_ref[...] + b_ref[...]
    cnt = jnp.maximum(a[:, HH : HH + 1], 1.0)
    mean = a[:, :HH] / cnt
    hroot = jnp.dot(h_ref[...], Wroot_ref[...], preferred_element_type=jnp.float32)
    hh = jnp.maximum(mean + hroot + bconv_ref[...], 0.0)
    o_ref[...] = (
        jnp.dot(hh, Wp2_ref[...], preferred_element_type=jnp.float32) + bp2_ref[...]
    )


def _finalize(agg0, agg1, h, Wroot, bconv, Wp2, bp2):
    return pl.pallas_call(
        _final_body,
        grid=(NN // NB,),
        in_specs=[
            pl.BlockSpec((NB, MSG_W), lambda i: (i, 0)),
            pl.BlockSpec((NB, MSG_W), lambda i: (i, 0)),
            pl.BlockSpec((NB, HH), lambda i: (i, 0)),
            pl.BlockSpec((HH, HH), lambda i: (0, 0)),
            pl.BlockSpec((1, HH), lambda i: (0, 0)),
            pl.BlockSpec((HH, NODE_OUT), lambda i: (0, 0)),
            pl.BlockSpec((1, NODE_OUT), lambda i: (0, 0)),
        ],
        out_specs=pl.BlockSpec((NB, NODE_OUT), lambda i: (i, 0)),
        out_shape=jax.ShapeDtypeStruct((NN, NODE_OUT), jnp.float32),
    )(agg0, agg1, h, Wroot, bconv.reshape(1, HH), Wp2, bp2.reshape(1, NODE_OUT))


def kernel(x, edge_index, edge_attr, Wp1, bp1, We1, be1, We2, be2, Wroot, bconv, Wp2, bp2):
    src = edge_index[0]
    dst = edge_index[1]
    npad = E_PAD - EE
    src2d = jnp.concatenate([src, jnp.zeros((npad,), jnp.int32)]).reshape(-1, IDXW)
    dst2d = jnp.concatenate([dst, jnp.full((npad,), NN, jnp.int32)]).reshape(-1, IDXW)
    ea_p = jnp.concatenate([edge_attr, jnp.zeros((npad, EDGE_IN), jnp.float32)])

    h = _node_proj1(x, Wp1, bp1)
    h_src = _sc_gather(h, src2d)
    msg = _edge_messages(ea_p, h_src, We1, be1, We2, be2)
    p0, p1 = _sc_scatter(msg, dst2d)
    return _finalize(p0, p1, h, Wroot, bconv, Wp2, bp2)


# speedup vs baseline: 5.0101x; 1.0514x over previous
import functools

import jax
import jax.numpy as jnp
from jax import lax
from jax.experimental import pallas as pl
from jax.experimental.pallas import tpu as pltpu
from jax.experimental.pallas import tpu_sc as plsc

NN = 10000
EE = 160000
HH = 32
NODE_IN = 128
EDGE_IN = 16
NODE_OUT = 128

NB = 1000
TE = 2048
E_PAD = 163840
MSG_W = 40
N_PAD = 10240

NC = 2
NS = 16
NW = NC * NS
IDXW = 128
ROWS_W = E_PAD // NW // IDXW
CHB = 8
CH = CHB * IDXW
NCHUNK = ROWS_W // CHB

def _sc_mesh():
    return plsc.VectorSubcoreMesh(
        core_axis_name="c", subcore_axis_name="s", num_cores=NC, num_subcores=NS
    )


def _gather_body(h_hbm, src_hbm, out_hbm, h_sh, idx_v, rows_v, sem):
    core = lax.axis_index("c")
    sub = lax.axis_index("s")
    wid = sub * NC + core
    @pl.when(sub < 10)
    def _():
        pltpu.sync_copy(h_hbm.at[pl.ds(sub * 1000, 1000)],
                        h_sh.at[pl.ds(sub * 1000, 1000)])

    plsc.subcore_barrier()
    for c in range(NCHUNK):
        row0 = wid * ROWS_W + c * CHB
        pltpu.sync_copy(src_hbm.at[pl.ds(row0, CHB)], idx_v)
        cps = [
            pltpu.async_copy(
                h_sh.at[idx_v.at[j]], rows_v.at[pl.ds(j * IDXW, IDXW)], sem
            )
            for j in range(CHB)
        ]
        for cp in cps:
            cp.wait()
        pltpu.sync_copy(rows_v, out_hbm.at[pl.ds(row0 * IDXW, CH)])


def _sc_gather(h, src2d):
    return pl.kernel(
        _gather_body,
        out_type=jax.ShapeDtypeStruct((E_PAD, HH), jnp.float32),
        mesh=_sc_mesh(),
        compiler_params=pltpu.CompilerParams(use_tc_tiling_on_sc=False),
        scratch_types=[
            pltpu.VMEM_SHARED((NN, HH), jnp.float32),
            pltpu.VMEM((CHB, IDXW), jnp.int32),
            pltpu.VMEM((CH, HH), jnp.float32),
            pltpu.SemaphoreType.DMA,
        ],
    )(h, src2d)


def _scatter_body(msg_hbm, dst_hbm, zero_hbm, out0_hbm, out1_hbm,
                  idx_v, rows_v, agg_sh, sem):
    core = lax.axis_index("c")
    sub = lax.axis_index("s")
    wid = sub * NC + core
    zrows = N_PAD // NS
    pltpu.sync_copy(zero_hbm.at[pl.ds(sub * zrows, zrows)],
                    agg_sh.at[pl.ds(sub * zrows, zrows)])
    plsc.subcore_barrier()
    for c in range(NCHUNK):
        row0 = wid * ROWS_W + c * CHB
        pltpu.sync_copy(dst_hbm.at[pl.ds(row0, CHB)], idx_v)
        pltpu.sync_copy(msg_hbm.at[pl.ds(row0 * IDXW, CH)], rows_v)
        cps = [
            pltpu.async_copy(
                rows_v.at[pl.ds(j * IDXW, IDXW)], agg_sh.at[idx_v.at[j]], sem,
                add=True,
            )
            for j in range(CHB)
        ]
        for cp in cps:
            cp.wait()
    plsc.subcore_barrier()

    @pl.when(core == 0)
    def _():
        pltpu.sync_copy(agg_sh.at[pl.ds(sub * zrows, zrows)],
                        out0_hbm.at[pl.ds(sub * zrows, zrows)])

    @pl.when(core == 1)
    def _():
        pltpu.sync_copy(agg_sh.at[pl.ds(sub * zrows, zrows)],
                        out1_hbm.at[pl.ds(sub * zrows, zrows)])


def _sc_scatter(msg, dst2d):
    zero = jnp.zeros((N_PAD, MSG_W), jnp.float32)
    return pl.kernel(
        _scatter_body,
        out_type=(
            jax.ShapeDtypeStruct((N_PAD, MSG_W), jnp.float32),
            jax.ShapeDtypeStruct((N_PAD, MSG_W), jnp.float32),
        ),
        mesh=_sc_mesh(),
        compiler_params=pltpu.CompilerParams(use_tc_tiling_on_sc=False),
        scratch_types=[
            pltpu.VMEM((CHB, IDXW), jnp.int32),
            pltpu.VMEM((CH, MSG_W), jnp.float32),
            pltpu.VMEM_SHARED((N_PAD, MSG_W), jnp.float32),
            pltpu.SemaphoreType.DMA,
        ],
    )(msg, dst2d, zero)


def _proj1_body(x_ref, w_ref, b_ref, o_ref):
    o_ref[...] = (
        jnp.dot(x_ref[...], w_ref[...], preferred_element_type=jnp.float32)
        + b_ref[...]
    )


def _node_proj1(x, Wp1, bp1):
    return pl.pallas_call(
        _proj1_body,
        grid=(NN // NB,),
        in_specs=[
            pl.BlockSpec((NB, NODE_IN), lambda i: (i, 0)),
            pl.BlockSpec((NODE_IN, HH), lambda i: (0, 0)),
            pl.BlockSpec((1, HH), lambda i: (0, 0)),
        ],
        out_specs=pl.BlockSpec((NB, HH), lambda i: (i, 0)),
        out_shape=jax.ShapeDtypeStruct((NN, HH), jnp.float32),
    )(x, Wp1, bp1.reshape(1, HH))


def _edge_body(ea_ref, hs_ref, We1_ref, be1_ref, We2_ref, be2_ref, rsel_ref, o_ref):
    eact = jnp.maximum(
        jnp.dot(ea_ref[...], We1_ref[...], preferred_element_type=jnp.float32)
        + be1_ref[...],
        0.0,
    )
    e_row = (
        jnp.dot(eact.astype(jnp.bfloat16), We2_ref[...],
                preferred_element_type=jnp.float32)
        + be2_ref[...]
    )
    w = jnp.dot(hs_ref[...].astype(jnp.bfloat16), rsel_ref[...],
                preferred_element_type=jnp.float32)
    acc = e_row[:, 0:128] * w[:, 0:128]
    for g in range(1, (HH * HH) // 128):
        acc = acc + e_row[:, g * 128 : (g + 1) * 128] * w[:, g * 128 : (g + 1) * 128]
    msg = acc[:, 0:HH] + acc[:, HH : 2 * HH] + acc[:, 2 * HH : 3 * HH] + acc[:, 3 * HH : 4 * HH]
    ones = jnp.ones((TE, 1), jnp.float32)
    zeros = jnp.zeros((TE, MSG_W - HH - 1), jnp.float32)
    o_ref[...] = jnp.concatenate([msg, ones, zeros], axis=1)


def _edge_messages(ea_pad, h_src, We1, be1, We2, be2):
    rsel = jnp.kron(jnp.eye(HH, dtype=jnp.bfloat16), jnp.ones((1, HH), jnp.bfloat16))
    We2 = We2.astype(jnp.bfloat16)
    return pl.pallas_call(
        _edge_body,
        grid=(E_PAD // TE,),
        in_specs=[
            pl.BlockSpec((TE, EDGE_IN), lambda i: (i, 0)),
            pl.BlockSpec((TE, HH), lambda i: (i, 0)),
            pl.BlockSpec((EDGE_IN, HH), lambda i: (0, 0)),
            pl.BlockSpec((1, HH), lambda i: (0, 0)),
            pl.BlockSpec((HH, HH * HH), lambda i: (0, 0)),
            pl.BlockSpec((1, HH * HH), lambda i: (0, 0)),
            pl.BlockSpec((HH, HH * HH), lambda i: (0, 0)),
        ],
        out_specs=pl.BlockSpec((TE, MSG_W), lambda i: (i, 0)),
        out_shape=jax.ShapeDtypeStruct((E_PAD, MSG_W), jnp.float32),
    )(ea_pad, h_src, We1, be1.reshape(1, HH), We2, be2.reshape(1, HH * HH), rsel)


def _final_body(a_ref, b_ref, h_ref, Wroot_ref, bconv_ref, Wp2_ref, bp2_ref, o_ref):
    a = a_ref[...] + b_ref[...]
    cnt = jnp.maximum(a[:, HH : HH + 1], 1.0)
    mean = a[:, :HH] / cnt
    hroot = jnp.dot(h_ref[...], Wroot_ref[...], preferred_element_type=jnp.float32)
    hh = jnp.maximum(mean + hroot + bconv_ref[...], 0.0)
    o_ref[...] = (
        jnp.dot(hh, Wp2_ref[...], preferred_element_type=jnp.float32) + bp2_ref[...]
    )


def _finalize(agg0, agg1, h, Wroot, bconv, Wp2, bp2):
    return pl.pallas_call(
        _final_body,
        grid=(NN // NB,),
        in_specs=[
            pl.BlockSpec((NB, MSG_W), lambda i: (i, 0)),
            pl.BlockSpec((NB, MSG_W), lambda i: (i, 0)),
            pl.BlockSpec((NB, HH), lambda i: (i, 0)),
            pl.BlockSpec((HH, HH), lambda i: (0, 0)),
            pl.BlockSpec((1, HH), lambda i: (0, 0)),
            pl.BlockSpec((HH, NODE_OUT), lambda i: (0, 0)),
            pl.BlockSpec((1, NODE_OUT), lambda i: (0, 0)),
        ],
        out_specs=pl.BlockSpec((NB, NODE_OUT), lambda i: (i, 0)),
        out_shape=jax.ShapeDtypeStruct((NN, NODE_OUT), jnp.float32),
    )(agg0, agg1, h, Wroot, bconv.reshape(1, HH), Wp2, bp2.reshape(1, NODE_OUT))


def kernel(x, edge_index, edge_attr, Wp1, bp1, We1, be1, We2, be2, Wroot, bconv, Wp2, bp2):
    src = edge_index[0]
    dst = edge_index[1]
    npad = E_PAD - EE
    src2d = jnp.concatenate([src, jnp.zeros((npad,), jnp.int32)]).reshape(-1, IDXW)
    dst2d = jnp.concatenate([dst, jnp.full((npad,), NN, jnp.int32)]).reshape(-1, IDXW)
    ea_p = jnp.concatenate([edge_attr, jnp.zeros((npad, EDGE_IN), jnp.float32)])

    h = _node_proj1(x, Wp1, bp1)
    h_src = _sc_gather(h, src2d)
    msg = _edge_messages(ea_p, h_src, We1, be1, We2, be2)
    p0, p1 = _sc_scatter(msg, dst2d)
    return _finalize(p0, p1, h, Wroot, bconv, Wp2, bp2)


# speedup vs baseline: 6.9984x; 1.3969x over previous
import functools

import jax
import jax.numpy as jnp
from jax import lax
from jax.experimental import pallas as pl
from jax.experimental.pallas import tpu as pltpu
from jax.experimental.pallas import tpu_sc as plsc

NN = 10000
EE = 160000
HH = 32
NODE_IN = 128
EDGE_IN = 16
NODE_OUT = 128

NB = 1000
TE = 2000
E_PAD = 163840
MSG_W = 40
IF_W = 128
N_PAD = 10240

NC = 2
NS = 16
NW = NC * NS
IDXW = 128
ROWS_W = E_PAD // NW // IDXW
CHB = 8
CH = CHB * IDXW
NCHUNK = ROWS_W // CHB


def _sc_mesh():
    return plsc.VectorSubcoreMesh(
        core_axis_name="c", subcore_axis_name="s", num_cores=NC, num_subcores=NS
    )


def _gather_body(h_hbm, src_hbm, out_hbm, h_sh, idx_v, rows_v, sem):
    core = lax.axis_index("c")
    sub = lax.axis_index("s")
    wid = sub * NC + core
    @pl.when(sub < 10)
    def _():
        pltpu.sync_copy(h_hbm.at[pl.ds(sub * 1000, 1000)],
                        h_sh.at[pl.ds(sub * 1000, 1000)])

    plsc.subcore_barrier()
    for c in range(NCHUNK):
        row0 = wid * ROWS_W + c * CHB
        pltpu.sync_copy(src_hbm.at[pl.ds(row0, CHB)], idx_v)
        cps = [
            pltpu.async_copy(
                h_sh.at[idx_v.at[j]], rows_v.at[pl.ds(j * IDXW, IDXW)], sem
            )
            for j in range(CHB)
        ]
        for cp in cps:
            cp.wait()
        pltpu.sync_copy(rows_v,
                        out_hbm.at[pl.ds(row0 * IDXW, CH), pl.ds(0, HH)])


def _sc_gather(h, src2d):
    return pl.kernel(
        _gather_body,
        out_type=jax.ShapeDtypeStruct((E_PAD, IF_W), jnp.float32),
        mesh=_sc_mesh(),
        compiler_params=pltpu.CompilerParams(use_tc_tiling_on_sc=False),
        scratch_types=[
            pltpu.VMEM_SHARED((NN, HH), jnp.float32),
            pltpu.VMEM((CHB, IDXW), jnp.int32),
            pltpu.VMEM((CH, HH), jnp.float32),
            pltpu.SemaphoreType.DMA,
        ],
    )(h, src2d)


def _scatter_body(msg_hbm, dst_hbm, zero_hbm, out0_hbm, out1_hbm,
                  idx_v, rows_v, agg_sh, sem):
    core = lax.axis_index("c")
    sub = lax.axis_index("s")
    wid = sub * NC + core
    zrows = N_PAD // NS
    pltpu.sync_copy(zero_hbm.at[pl.ds(sub * zrows, zrows)],
                    agg_sh.at[pl.ds(sub * zrows, zrows)])
    plsc.subcore_barrier()
    for c in range(NCHUNK):
        row0 = wid * ROWS_W + c * CHB
        pltpu.sync_copy(dst_hbm.at[pl.ds(row0, CHB)], idx_v)
        pltpu.sync_copy(msg_hbm.at[pl.ds(row0 * IDXW, CH), pl.ds(0, MSG_W)],
                        rows_v)
        cps = [
            pltpu.async_copy(
                rows_v.at[pl.ds(j * IDXW, IDXW)], agg_sh.at[idx_v.at[j]], sem,
                add=True,
            )
            for j in range(CHB)
        ]
        for cp in cps:
            cp.wait()
    plsc.subcore_barrier()

    @pl.when(core == 0)
    def _():
        pltpu.sync_copy(agg_sh.at[pl.ds(sub * zrows, zrows)],
                        out0_hbm.at[pl.ds(sub * zrows, zrows)])

    @pl.when(core == 1)
    def _():
        pltpu.sync_copy(agg_sh.at[pl.ds(sub * zrows, zrows)],
                        out1_hbm.at[pl.ds(sub * zrows, zrows)])


def _sc_scatter(msg, dst2d):
    zero = jnp.zeros((N_PAD, MSG_W), jnp.float32)
    return pl.kernel(
        _scatter_body,
        out_type=(
            jax.ShapeDtypeStruct((N_PAD, MSG_W), jnp.float32),
            jax.ShapeDtypeStruct((N_PAD, MSG_W), jnp.float32),
        ),
        mesh=_sc_mesh(),
        compiler_params=pltpu.CompilerParams(use_tc_tiling_on_sc=False),
        scratch_types=[
            pltpu.VMEM((CHB, IDXW), jnp.int32),
            pltpu.VMEM((CH, MSG_W), jnp.float32),
            pltpu.VMEM_SHARED((N_PAD, MSG_W), jnp.float32),
            pltpu.SemaphoreType.DMA,
        ],
    )(msg, dst2d, zero)


def _proj1_body(x_ref, w_ref, b_ref, o_ref):
    o_ref[...] = (
        jnp.dot(x_ref[...], w_ref[...], preferred_element_type=jnp.float32)
        + b_ref[...]
    )


def _node_proj1(x, Wp1, bp1):
    return pl.pallas_call(
        _proj1_body,
        grid=(NN // NB,),
        in_specs=[
            pl.BlockSpec((NB, NODE_IN), lambda i: (i, 0)),
            pl.BlockSpec((NODE_IN, HH), lambda i: (0, 0)),
            pl.BlockSpec((1, HH), lambda i: (0, 0)),
        ],
        out_specs=pl.BlockSpec((NB, HH), lambda i: (i, 0)),
        out_shape=jax.ShapeDtypeStruct((NN, HH), jnp.float32),
    )(x, Wp1, bp1.reshape(1, HH))


def _edge_body(ea_ref, hs_ref, We1_ref, be1_ref, We2_ref, be2_ref, rsel_ref, o_ref):
    eact = jnp.maximum(
        jnp.dot(ea_ref[...], We1_ref[...], preferred_element_type=jnp.float32)
        + be1_ref[...],
        0.0,
    )
    e_row = (
        jnp.dot(eact.astype(jnp.bfloat16), We2_ref[...],
                preferred_element_type=jnp.float32)
        + be2_ref[...]
    )
    hs = hs_ref[...][:, 0:HH]
    w = jnp.dot(hs.astype(jnp.bfloat16), rsel_ref[...],
                preferred_element_type=jnp.float32)
    acc = e_row[:, 0:128] * w[:, 0:128]
    for g in range(1, (HH * HH) // 128):
        acc = acc + e_row[:, g * 128 : (g + 1) * 128] * w[:, g * 128 : (g + 1) * 128]
    msg = acc[:, 0:HH] + acc[:, HH : 2 * HH] + acc[:, 2 * HH : 3 * HH] + acc[:, 3 * HH : 4 * HH]
    ones = jnp.ones((TE, 1), jnp.float32)
    zeros = jnp.zeros((TE, IF_W - HH - 1), jnp.float32)
    o_ref[...] = jnp.concatenate([msg, ones, zeros], axis=1)


def _edge_messages(edge_attr, h_src, We1, be1, We2, be2):
    rsel = jnp.kron(jnp.eye(HH, dtype=jnp.bfloat16), jnp.ones((1, HH), jnp.bfloat16))
    We2 = We2.astype(jnp.bfloat16)
    return pl.pallas_call(
        _edge_body,
        grid=(EE // TE,),
        in_specs=[
            pl.BlockSpec((TE, EDGE_IN), lambda i: (i, 0)),
            pl.BlockSpec((TE, IF_W), lambda i: (i, 0)),
            pl.BlockSpec((EDGE_IN, HH), lambda i: (0, 0)),
            pl.BlockSpec((1, HH), lambda i: (0, 0)),
            pl.BlockSpec((HH, HH * HH), lambda i: (0, 0)),
            pl.BlockSpec((1, HH * HH), lambda i: (0, 0)),
            pl.BlockSpec((HH, HH * HH), lambda i: (0, 0)),
        ],
        out_specs=pl.BlockSpec((TE, IF_W), lambda i: (i, 0)),
        out_shape=jax.ShapeDtypeStruct((E_PAD, IF_W), jnp.float32),
    )(edge_attr, h_src, We1, be1.reshape(1, HH), We2, be2.reshape(1, HH * HH), rsel)


def _final_body(a_ref, b_ref, h_ref, Wroot_ref, bconv_ref, Wp2_ref, bp2_ref, o_ref):
    a = a_ref[...] + b_ref[...]
    cnt = jnp.maximum(a[:, HH : HH + 1], 1.0)
    mean = a[:, :HH] / cnt
    hroot = jnp.dot(h_ref[...], Wroot_ref[...],
                    preferred_element_type=jnp.float32)
    hh = jnp.maximum(mean + hroot + bconv_ref[...], 0.0)
    o_ref[...] = (
        jnp.dot(hh, Wp2_ref[...], preferred_element_type=jnp.float32) + bp2_ref[...]
    )


def _finalize(agg0, agg1, h, Wroot, bconv, Wp2, bp2):
    return pl.pallas_call(
        _final_body,
        grid=(NN // NB,),
        in_specs=[
            pl.BlockSpec((NB, MSG_W), lambda i: (i, 0)),
            pl.BlockSpec((NB, MSG_W), lambda i: (i, 0)),
            pl.BlockSpec((NB, HH), lambda i: (i, 0)),
            pl.BlockSpec((HH, HH), lambda i: (0, 0)),
            pl.BlockSpec((1, HH), lambda i: (0, 0)),
            pl.BlockSpec((HH, NODE_OUT), lambda i: (0, 0)),
            pl.BlockSpec((1, NODE_OUT), lambda i: (0, 0)),
        ],
        out_specs=pl.BlockSpec((NB, NODE_OUT), lambda i: (i, 0)),
        out_shape=jax.ShapeDtypeStruct((NN, NODE_OUT), jnp.float32),
    )(agg0, agg1, h, Wroot, bconv.reshape(1, HH), Wp2, bp2.reshape(1, NODE_OUT))


def kernel(x, edge_index, edge_attr, Wp1, bp1, We1, be1, We2, be2, Wroot, bconv, Wp2, bp2):
    src = edge_index[0]
    dst = edge_index[1]
    npad = E_PAD - EE
    src2d = jnp.concatenate([src, jnp.zeros((npad,), jnp.int32)]).reshape(-1, IDXW)
    dst2d = jnp.concatenate([dst, jnp.full((npad,), NN, jnp.int32)]).reshape(-1, IDXW)

    h = _node_proj1(x, Wp1, bp1)
    h_src = _sc_gather(h, src2d)
    msg = _edge_messages(edge_attr, h_src, We1, be1, We2, be2)
    p0, p1 = _sc_scatter(msg, dst2d)
    return _finalize(p0, p1, h, Wroot, bconv, Wp2, bp2)
